# baseline (speedup 1.0000x reference)
import functools

import jax
import jax.numpy as jnp
from jax import lax
from jax.experimental import pallas as pl
from jax.experimental.pallas import tpu as pltpu
from jax.experimental.pallas import tpu_sc as plsc

N_CHARS = 512
EMB_DIM = 128
T = 16
L = 16
HALF = EMB_DIM // 2


def _build(n_words):
    info = plsc.get_sparse_core_info()
    nc, ns = info.num_cores, info.num_subcores
    nw = nc * ns
    W = n_words // nw
    n_groups = W // L
    mesh = plsc.VectorSubcoreMesh(core_axis_name="c", subcore_axis_name="s")

    @functools.partial(
        pl.kernel,
        mesh=mesh,
        compiler_params=pltpu.CompilerParams(
            needs_layout_passes=False, use_tc_tiling_on_sc=False),
        out_type=jax.ShapeDtypeStruct((n_words, EMB_DIM), jnp.float32),
        scratch_types=[
            pltpu.VMEM((N_CHARS * HALF,), jnp.int32),
            pltpu.VMEM((W, T), jnp.int32),
            pltpu.VMEM((W,), jnp.int32),
            pltpu.VMEM((W + L,), jnp.int32),
            pltpu.VMEM((W, EMB_DIM), jnp.float32),
        ],
    )
    def k(ci_hbm, ln_hbm, emb_hbm, out_hbm, table_v, chars_v, lens_v,
          sidx_v, outf_v):
        wid = lax.axis_index("s") * nc + lax.axis_index("c")
        base = wid * W
        pltpu.sync_copy(emb_hbm, table_v)
        pltpu.sync_copy(ci_hbm.at[pl.ds(base, W)], chars_v)
        pltpu.sync_copy(ln_hbm.at[pl.ds(base, W)], lens_v)

        iota = lax.iota(jnp.int32, L)
        cols = [iota + k0 * L for k0 in range(HALF // L)]
        ocols = [iota + k0 * L for k0 in range(EMB_DIM // L)]
        zero = jnp.zeros((L,), jnp.float32)

        def count_body(g, cnts):
            lv = lens_v[pl.ds(g * L, L)]
            new = []
            for l0 in range(T + 1):
                m = lv == l0
                c = plsc.all_reduce_population_count(m)
                new.append(cnts[l0] + jnp.max(c, axis=0))
            return tuple(new)

        counts = lax.fori_loop(
            0, n_groups, count_body,
            tuple(jnp.int32(0) for _ in range(T + 1)))

        starts = []
        acc = jnp.int32(0)
        for l0 in range(T + 1):
            starts.append(acc)
            acc = acc + counts[l0]

        def fill_body(g, curs):
            lv = lens_v[pl.ds(g * L, L)]
            widx = iota + g * L
            new = []
            for l0 in range(T + 1):
                m = lv == l0
                rank = plsc.cumsum(m.astype(jnp.int32)) - 1
                plsc.store_scatter(sidx_v, [curs[l0] + rank], widx, mask=m)
                c = plsc.all_reduce_population_count(m)
                new.append(curs[l0] + jnp.max(c, axis=0))
            return tuple(new)

        lax.fori_loop(0, n_groups, fill_body, tuple(starts))

        def zero_body(i, carry):
            posf = jnp.full((L,), starts[0] + i, jnp.int32)
            wf = plsc.load_gather(sidx_v, [posf])
            for k0 in range(EMB_DIM // L):
                plsc.store_scatter(outf_v, [wf, ocols[k0]], zero)
            return carry

        lax.fori_loop(0, counts[0], zero_body, 0)

        for l0 in range(1, T + 1):
            inv = jnp.float32(1.0 / l0)

            def len_body(i, carry, l0=l0, inv=inv):
                posf = jnp.full((L,), starts[l0] + i, jnp.int32)
                wf = plsc.load_gather(sidx_v, [posf])
                cv = plsc.load_gather(chars_v, [wf, iota])
                sh = cv << 6
                accs = [zero] * (EMB_DIM // L)
                for t in range(l0):
                    spl = jnp.full((L,), sh[t], jnp.int32)
                    for k0 in range(HALF // L):
                        g = plsc.load_gather(table_v, [spl + cols[k0]])
                        lo, hi = plsc.unpack(
                            plsc.bitcast(g, jnp.bfloat16),
                            format=plsc.PackFormat.INTERLEAVED)
                        accs[k0] = accs[k0] + lo
                        accs[k0 + 4] = accs[k0 + 4] + hi
                for k0 in range(EMB_DIM // L):
                    plsc.store_scatter(
                        outf_v, [wf, ocols[k0]], accs[k0] * inv)
                return carry

            lax.fori_loop(0, counts[l0], len_body, 0)

        pltpu.sync_copy(outf_v, out_hbm.at[pl.ds(base, W)])

    return k


def kernel(char_input, lengths, embedding):
    b, s, t = char_input.shape
    n = b * s
    ci = char_input.reshape(n, t)
    ln = lengths.reshape(n)
    emb_bf = embedding.astype(jnp.bfloat16)
    packed = jax.lax.bitcast_convert_type(
        jnp.stack([emb_bf[:, :HALF], emb_bf[:, HALF:]], axis=-1), jnp.int32)
    out = _build(n)(ci, ln, packed.reshape(-1))
    return out.reshape(b, s, EMB_DIM), ln
